# baseline (speedup 1.0000x reference)
import functools

import jax
import jax.numpy as jnp
from jax import lax
from jax.experimental import pallas as pl
from jax.experimental.pallas import tpu as pltpu
from jax.experimental.pallas import tpu_sc as plsc

B = 128
N = 32768
M = N // 2

_info = plsc.get_sparse_core_info()
_NC, _NS, _L = _info.num_cores, _info.num_subcores, _info.num_lanes
_NW = _NC * _NS
_GROUPS = B // 8
_HALF = N // 2
CHUNK = 4096
_NCH = _HALF // CHUNK


def _sc_body(in_hbm, out_hbm, in_v0, in_v1, out_v0, out_v1,
             in_sem0, in_sem1, out_sem0, out_sem1):
    wid = lax.axis_index("s") * _NC + lax.axis_index("c")
    g = wid // 2
    half = wid % 2
    row0 = pl.multiple_of(g * 8, 8)
    col0 = pl.multiple_of(half * _HALF, _HALF)
    lane = lax.iota(jnp.int32, _L)
    in_bufs = (in_v0, in_v1)
    out_bufs = (out_v0, out_v1)
    in_sems = (in_sem0, in_sem1)
    out_sems = (out_sem0, out_sem1)

    def src(c):
        off = pl.multiple_of(col0 + c * CHUNK, 128)
        return in_hbm.at[pl.ds(row0, 8), pl.ds(off, CHUNK)]

    def dst(c):
        off = pl.multiple_of((col0 + c * CHUNK) // 2, 128)
        return out_hbm.at[pl.ds(row0, 8), pl.ds(off, CHUNK // 2)]

    def gather(sv, dv):
        for r in range(8):
            rv = jnp.full((_L,), r, jnp.int32)

            @plsc.parallel_loop(0, CHUNK // 2 // _L, unroll=8)
            def _(j, _rv=rv):
                jv = j * _L + lane
                x = plsc.load_gather(sv, [_rv, 2 * jv])
                plsc.store_scatter(dv, [_rv, jv], x)

    in_cp = [None] * _NCH
    out_cp = [None] * _NCH
    in_cp[0] = pltpu.async_copy(src(0), in_bufs[0], in_sems[0])
    for c in range(_NCH):
        p = c % 2
        in_cp[c].wait()
        if c + 1 < _NCH:
            in_cp[c + 1] = pltpu.async_copy(src(c + 1), in_bufs[1 - p],
                                            in_sems[1 - p])
        if c >= 2:
            out_cp[c - 2].wait()
        gather(in_bufs[p], out_bufs[p])
        out_cp[c] = pltpu.async_copy(out_bufs[p], dst(c), out_sems[p])
    out_cp[_NCH - 2].wait()
    out_cp[_NCH - 1].wait()


@jax.jit
def kernel(inputs):
    mesh = plsc.VectorSubcoreMesh(core_axis_name="c", subcore_axis_name="s")
    f = functools.partial(
        pl.kernel,
        mesh=mesh,
        out_type=jax.ShapeDtypeStruct((B, M), jnp.float32),
        scratch_types=[
            pltpu.VMEM((8, CHUNK), jnp.float32),
            pltpu.VMEM((8, CHUNK), jnp.float32),
            pltpu.VMEM((8, CHUNK // 2), jnp.float32),
            pltpu.VMEM((8, CHUNK // 2), jnp.float32),
            pltpu.SemaphoreType.DMA,
            pltpu.SemaphoreType.DMA,
            pltpu.SemaphoreType.DMA,
            pltpu.SemaphoreType.DMA,
        ],
        compiler_params=pltpu.CompilerParams(
            needs_layout_passes=False, use_tc_tiling_on_sc=True),
    )(_sc_body)
    return f(inputs)


# speedup vs baseline: 1.1769x; 1.1769x over previous
import functools

import jax
import jax.numpy as jnp
from jax import lax
from jax.experimental import pallas as pl
from jax.experimental.pallas import tpu as pltpu
from jax.experimental.pallas import tpu_sc as plsc

B = 128
N = 32768
M = N // 2

_info = plsc.get_sparse_core_info()
_NC, _NS, _L = _info.num_cores, _info.num_subcores, _info.num_lanes
_NW = _NC * _NS
_ROWS_PER_W = B // _NW

CHUNK = N // 2
_CPR = N // CHUNK
_NCH = _ROWS_PER_W * _CPR
RING = 3


def _sc_body(in_hbm, out_hbm, *scratch):
    in_bufs = scratch[0:RING]
    out_bufs = scratch[RING:2 * RING]
    in_sems = scratch[2 * RING:3 * RING]
    out_sems = scratch[3 * RING:4 * RING]

    wid = lax.axis_index("s") * _NC + lax.axis_index("c")
    base_row = wid * _ROWS_PER_W
    lane = lax.iota(jnp.int32, _L)

    def chunk_src(c):
        row = base_row + c // _CPR
        return in_hbm.at[row, pl.ds((c % _CPR) * CHUNK, CHUNK)]

    def chunk_dst(c):
        row = base_row + c // _CPR
        return out_hbm.at[row, pl.ds((c % _CPR) * (CHUNK // 2), CHUNK // 2)]

    def gather(src, dst):
        @plsc.parallel_loop(0, CHUNK // 2 // _L, unroll=8)
        def _(j):
            idx = (2 * _L) * j + 2 * lane
            dst[pl.ds(j * _L, _L)] = plsc.load_gather(src, [idx])

    in_cp = {}
    out_cp = {}
    for c in range(RING - 1):
        in_cp[c] = pltpu.async_copy(chunk_src(c), in_bufs[c % RING],
                                    in_sems[c % RING])
    for c in range(_NCH):
        p = c % RING
        in_cp[c].wait()
        nxt = c + RING - 1
        if nxt < _NCH:
            in_cp[nxt] = pltpu.async_copy(chunk_src(nxt), in_bufs[nxt % RING],
                                          in_sems[nxt % RING])
        if c >= RING:
            out_cp[c - RING].wait()
        gather(in_bufs[p], out_bufs[p])
        out_cp[c] = pltpu.async_copy(out_bufs[p], chunk_dst(c), out_sems[p])
    for c in range(_NCH - RING, _NCH):
        out_cp[c].wait()


@jax.jit
def kernel(inputs):
    mesh = plsc.VectorSubcoreMesh(core_axis_name="c", subcore_axis_name="s")
    f = functools.partial(
        pl.kernel,
        mesh=mesh,
        out_type=jax.ShapeDtypeStruct((B, M), jnp.float32),
        scratch_types=(
            [pltpu.VMEM((CHUNK,), jnp.float32) for _ in range(RING)]
            + [pltpu.VMEM((CHUNK // 2,), jnp.float32) for _ in range(RING)]
            + [pltpu.SemaphoreType.DMA for _ in range(2 * RING)]
        ),
        compiler_params=pltpu.CompilerParams(needs_layout_passes=False),
    )(_sc_body)
    return f(inputs)


# speedup vs baseline: 1.1870x; 1.0086x over previous
import functools

import jax
import jax.numpy as jnp
from jax import lax
from jax.experimental import pallas as pl
from jax.experimental.pallas import tpu as pltpu
from jax.experimental.pallas import tpu_sc as plsc

B = 128
N = 32768
M = N // 2

_info = plsc.get_sparse_core_info()
_NC, _NS, _L = _info.num_cores, _info.num_subcores, _info.num_lanes
_NW = _NC * _NS
_ROWS_PER_W = B // _NW

CHUNK = N // 2
_CPR = N // CHUNK
_NCH = _ROWS_PER_W * _CPR
RING = 4


def _sc_body(in_hbm, out_hbm, *scratch):
    in_bufs = scratch[0:RING]
    out_bufs = scratch[RING:2 * RING]
    in_sems = scratch[2 * RING:3 * RING]
    out_sems = scratch[3 * RING:4 * RING]

    wid = lax.axis_index("s") * _NC + lax.axis_index("c")
    base_row = wid * _ROWS_PER_W
    lane = lax.iota(jnp.int32, _L)

    def chunk_src(c):
        row = base_row + c // _CPR
        return in_hbm.at[row, pl.ds((c % _CPR) * CHUNK, CHUNK)]

    def chunk_dst(c):
        row = base_row + c // _CPR
        return out_hbm.at[row, pl.ds((c % _CPR) * (CHUNK // 2), CHUNK // 2)]

    def gather(src, dst):
        @plsc.parallel_loop(0, CHUNK // 2 // _L, unroll=8)
        def _(j):
            idx = (2 * _L) * j + 2 * lane
            dst[pl.ds(j * _L, _L)] = plsc.load_gather(src, [idx])

    in_cp = {}
    out_cp = {}
    for c in range(RING - 1):
        in_cp[c] = pltpu.async_copy(chunk_src(c), in_bufs[c % RING],
                                    in_sems[c % RING])
    for c in range(_NCH):
        p = c % RING
        in_cp[c].wait()
        nxt = c + RING - 1
        if nxt < _NCH:
            in_cp[nxt] = pltpu.async_copy(chunk_src(nxt), in_bufs[nxt % RING],
                                          in_sems[nxt % RING])
        if c >= RING:
            out_cp[c - RING].wait()
        gather(in_bufs[p], out_bufs[p])
        out_cp[c] = pltpu.async_copy(out_bufs[p], chunk_dst(c), out_sems[p])
    for c in range(_NCH - RING, _NCH):
        out_cp[c].wait()


@jax.jit
def kernel(inputs):
    mesh = plsc.VectorSubcoreMesh(core_axis_name="c", subcore_axis_name="s")
    f = functools.partial(
        pl.kernel,
        mesh=mesh,
        out_type=jax.ShapeDtypeStruct((B, M), jnp.float32),
        scratch_types=(
            [pltpu.VMEM((CHUNK,), jnp.float32) for _ in range(RING)]
            + [pltpu.VMEM((CHUNK // 2,), jnp.float32) for _ in range(RING)]
            + [pltpu.SemaphoreType.DMA for _ in range(2 * RING)]
        ),
        compiler_params=pltpu.CompilerParams(needs_layout_passes=False),
    )(_sc_body)
    return f(inputs)
